# baseline (speedup 1.0000x reference)
import functools

import numpy as np
import jax
import jax.numpy as jnp
from jax import lax
from jax.experimental import pallas as pl
from jax.experimental.pallas import tpu as pltpu
from jax.experimental.pallas import tpu_sc as plsc

_NUM_CLASSES = 1000000
_E = 64
_NUM_SAMPLE = 10000
_BATCH = 1024
_S = 64.0
_COS_M = float(np.cos(0.5))
_SIN_M = float(np.sin(0.5))
_NEG_LOG_CLIP = float(-np.log(1e-30))

_N_CPAD = 10240
_LBL_OFF = 10240
_ZERO_COL = 11264
_TOTAL = 12288
_CHUNK = 1024
_NSTEPS = _TOTAL // _CHUNK


def _np_threefry2x32(keypair, x0, x1):
    rot1 = (13, 15, 26, 6)
    rot2 = (17, 29, 16, 24)
    ks0, ks1 = keypair
    ks2 = np.uint32(ks0 ^ ks1 ^ np.uint32(0x1BD11BDA))
    x0 = (x0 + ks0).astype(np.uint32)
    x1 = (x1 + ks1).astype(np.uint32)

    def rotl(v, d):
        return ((v << np.uint32(d)) | (v >> np.uint32(32 - d))).astype(np.uint32)

    ks = [ks1, ks2, ks0, ks1, ks2, ks0]
    for r in range(5):
        for d in rot1 if r % 2 == 0 else rot2:
            x0 = (x0 + x1).astype(np.uint32)
            x1 = rotl(x1, d)
            x1 = (x1 ^ x0).astype(np.uint32)
        x0 = (x0 + ks[r]).astype(np.uint32)
        x1 = (x1 + ks[r + 1] + np.uint32(r + 1)).astype(np.uint32)
    return x0, x1


def _np_uniform_01(seed, n):
    key = (np.uint32((seed >> 32) & 0xFFFFFFFF), np.uint32(seed & 0xFFFFFFFF))
    i = np.arange(n, dtype=np.uint64)
    c1 = (i >> np.uint64(32)).astype(np.uint32)
    c2 = (i & np.uint64(0xFFFFFFFF)).astype(np.uint32)
    o0, o1 = _np_threefry2x32(key, c1, c2)
    bits = o0 ^ o1
    f = (((bits >> np.uint32(9)) | np.uint32(0x3F800000)).view(np.float32)
         - np.float32(1.0))
    return np.maximum(np.float32(0.0), f)


def _cand_indices() -> np.ndarray:
    perm = _np_uniform_01(42, _NUM_CLASSES)
    order = np.lexsort((np.arange(_NUM_CLASSES), -perm.astype(np.float64)))
    return order[:_NUM_SAMPLE].astype(np.int32)


_CAND = _cand_indices()

_CLASS_ID_TMPL = np.full((_TOTAL,), -1, np.int32)
_CLASS_ID_TMPL[:_NUM_SAMPLE] = _CAND
_CLASS_ID_TMPL[_ZERO_COL] = 0

_GIDX_TMPL = np.zeros((_TOTAL,), np.int32)
_GIDX_TMPL[:_NUM_SAMPLE] = _CAND
_GIDX_TMPL[_ZERO_COL] = 0

_NW = 32
_BPW = _TOTAL // _NW
_GCHUNK = 128
_NGC = _BPW // _GCHUNK
_PAIR_ROWS = _NUM_CLASSES // 2
_PAIR_W = 2 * _E


def _sc_gather_kernel(table_hbm, idx_hbm, out_hbm, idx_v, rows_v, sem):
    wid = lax.axis_index("s") * 2 + lax.axis_index("c")
    base = wid * _BPW
    for j in range(_NGC):
        pltpu.sync_copy(idx_hbm.at[pl.ds(base + j * _GCHUNK, _GCHUNK)],
                        idx_v.at[j])
    copies = []
    for j in range(_NGC):
        copies.append(
            pltpu.async_copy(
                table_hbm.at[idx_v.at[j]],
                rows_v.at[pl.ds(j * _GCHUNK, _GCHUNK)],
                sem,
            ))
    for c in copies:
        c.wait()
    pltpu.sync_copy(rows_v, out_hbm.at[pl.ds(base, _BPW)])


def _sc_gather(table_pairs, pair_idx):
    mesh = plsc.VectorSubcoreMesh(core_axis_name="c", subcore_axis_name="s")
    kern = functools.partial(
        pl.kernel,
        mesh=mesh,
        out_type=jax.ShapeDtypeStruct((_TOTAL, _PAIR_W), jnp.float32),
        scratch_types=[
            pltpu.VMEM((_NGC, _GCHUNK), jnp.int32),
            pltpu.VMEM((_BPW, _PAIR_W), jnp.float32),
            pltpu.SemaphoreType.DMA,
        ],
    )(_sc_gather_kernel)
    return kern(table_pairs, pair_idx)


def _tc_body(emb_ref, w_ref, half_ref, cid_ref, labc_ref, labr_ref, out_ref,
             z_acc, marg, cnt):
    pid = pl.program_id(0)

    @pl.when(pid == 0)
    def _init():
        cnt[0, 0] = 0
        z_acc[...] = jnp.zeros_like(z_acc)

    labs_c = labc_ref[...]
    labs_r = labr_ref[...]
    cid = cid_ref[0]

    row_i = lax.broadcasted_iota(jnp.int32, (_BATCH, _CHUNK), 0)
    col_j = lax.broadcasted_iota(jnp.int32, (_BATCH, _CHUNK), 1)

    eq_ll = labs_c == labs_r
    dup_l = jnp.sum(jnp.where(eq_ll & (row_i < col_j), 1, 0),
                    axis=0, keepdims=True) > 0
    d = _BATCH - jnp.sum(dup_l.astype(jnp.int32))
    has0 = jnp.sum(jnp.where(labs_r == 0, 1, 0)) > 0
    include0 = jnp.logical_and(d < _BATCH, jnp.logical_not(has0))
    n_pos = d + include0.astype(jnp.int32)
    k_neg = _NUM_SAMPLE - n_pos

    gcol = pid * _CHUNK + lax.broadcasted_iota(jnp.int32, (1, _CHUNK), 1)
    region_c = gcol < _NUM_SAMPLE
    region_l = jnp.logical_and(gcol >= _LBL_OFF, gcol < _ZERO_COL)
    region_0 = gcol == _ZERO_COL

    eq = labs_c == cid
    in_p = jnp.sum(jnp.where(eq, 1, 0), axis=0, keepdims=True) > 0
    in_p = in_p | ((cid == 0) & include0)
    nonpos = (cid >= 0) & jnp.logical_not(in_p) & region_c
    npf = nonpos.astype(jnp.float32)
    tri = (row_i <= col_j).astype(jnp.float32)
    prefix_inc = lax.dot_general(npf, tri, (((1,), (0,)), ((), ())),
                                 preferred_element_type=jnp.float32)
    prefix_exc = prefix_inc - npf
    base = cnt[0, 0]
    sel = nonpos & ((base.astype(jnp.float32) + prefix_exc)
                    < k_neg.astype(jnp.float32))
    cnt[0, 0] = base + jnp.sum(npf).astype(jnp.int32)

    lbl_pos = gcol - _LBL_OFF
    dup_here = jnp.sum(jnp.where(eq & (row_i < lbl_pos), 1, 0),
                       axis=0, keepdims=True) > 0
    occ = jnp.logical_not(dup_here)

    colmask = sel | (region_l & occ) | (region_0 & (cid == 0) & include0)

    emb = emb_ref[...]
    en = jnp.sqrt(jnp.sum(emb * emb, axis=1, keepdims=True))
    nemb = emb / jnp.clip(en, 1e-12, None)
    wpair = w_ref[...]
    half = half_ref[...] > 0
    w = jnp.where(half, wpair[:, _E:], wpair[:, :_E])
    wn0 = jnp.sqrt(jnp.sum(w * w, axis=1, keepdims=True))
    wn = w / jnp.clip(wn0, 1e-12, None)

    logit = lax.dot_general(nemb, wn, (((1,), (1,)), ((), ())),
                            preferred_element_type=jnp.float32)
    logit = jnp.clip(logit, -1.0, 1.0)

    t = jnp.sum(nemb * wn, axis=1, keepdims=True)
    tcl = jnp.clip(jnp.clip(t, -1.0, 1.0), -1.0 + 1e-7, 1.0 - 1e-7)
    mrg = tcl * _COS_M - jnp.sqrt(1.0 - tcl * tcl) * _SIN_M

    @pl.when(pid == _LBL_OFF // _CHUNK)
    def _save_margin():
        marg[...] = mrg

    repl = eq & region_l & colmask
    ex = jnp.exp(_S * jnp.where(repl, mrg, logit))
    exm = jnp.where(colmask, ex, 0.0)
    z_acc[...] += jnp.sum(exm, axis=1, keepdims=True)

    @pl.when(pid == _NSTEPS - 1)
    def _finish():
        z = z_acc[...]
        m = marg[...]
        loss_vec = jnp.minimum(jnp.log(z) - _S * m, _NEG_LOG_CLIP)
        out_ref[...] = (jnp.sum(loss_vec) / float(_BATCH)).reshape(1, 1)


def _tc_compute(emb, rows, halfsel, class_id, labels):
    cid3 = class_id.reshape(_NSTEPS, 1, _CHUNK)
    half2 = halfsel.reshape(_TOTAL, 1)
    labs_c = labels.reshape(_BATCH, 1)
    labs_r = labels.reshape(1, _BATCH)
    return pl.pallas_call(
        _tc_body,
        grid=(_NSTEPS,),
        in_specs=[
            pl.BlockSpec((_BATCH, _E), lambda i: (0, 0)),
            pl.BlockSpec((_CHUNK, _PAIR_W), lambda i: (i, 0)),
            pl.BlockSpec((_CHUNK, 1), lambda i: (i, 0)),
            pl.BlockSpec((1, 1, _CHUNK), lambda i: (i, 0, 0)),
            pl.BlockSpec((_BATCH, 1), lambda i: (0, 0)),
            pl.BlockSpec((1, _BATCH), lambda i: (0, 0)),
        ],
        out_specs=pl.BlockSpec((1, 1), lambda i: (0, 0)),
        out_shape=jax.ShapeDtypeStruct((1, 1), jnp.float32),
        scratch_shapes=[
            pltpu.VMEM((_BATCH, 1), jnp.float32),
            pltpu.VMEM((_BATCH, 1), jnp.float32),
            pltpu.SMEM((1, 1), jnp.int32),
        ],
    )(emb, rows, half2, cid3, labs_c, labs_r)


def kernel(local_embeddings, local_labels, weight):
    labels = local_labels.astype(jnp.int32)
    class_id = jnp.asarray(_CLASS_ID_TMPL).at[_LBL_OFF:_ZERO_COL].set(labels)
    gidx = jnp.asarray(_GIDX_TMPL).at[_LBL_OFF:_ZERO_COL].set(labels)
    pair_idx = gidx % _PAIR_ROWS
    halfsel = gidx // _PAIR_ROWS
    table_pairs = jnp.concatenate(
        [weight[:_PAIR_ROWS], weight[_PAIR_ROWS:]], axis=1)
    rows = _sc_gather(table_pairs, pair_idx)
    out = _tc_compute(local_embeddings, rows, halfsel, class_id, labels)
    return out[0, 0]


# speedup vs baseline: 1.8700x; 1.8700x over previous
import functools

import numpy as np
import jax
import jax.numpy as jnp
from jax import lax
from jax.experimental import pallas as pl
from jax.experimental.pallas import tpu as pltpu
from jax.experimental.pallas import tpu_sc as plsc

_NUM_CLASSES = 1000000
_E = 64
_NUM_SAMPLE = 10000
_BATCH = 1024
_S = 64.0
_COS_M = float(np.cos(0.5))
_SIN_M = float(np.sin(0.5))
_NEG_LOG_CLIP = float(-np.log(1e-30))

_N_CPAD = 10240
_LBL_OFF = 10240
_ZERO_COL = 11264
_TOTAL = 12288
_CHUNK = 1024
_NSTEPS = _TOTAL // _CHUNK


def _np_threefry2x32(keypair, x0, x1):
    rot1 = (13, 15, 26, 6)
    rot2 = (17, 29, 16, 24)
    ks0, ks1 = keypair
    ks2 = np.uint32(ks0 ^ ks1 ^ np.uint32(0x1BD11BDA))
    x0 = (x0 + ks0).astype(np.uint32)
    x1 = (x1 + ks1).astype(np.uint32)

    def rotl(v, d):
        return ((v << np.uint32(d)) | (v >> np.uint32(32 - d))).astype(np.uint32)

    ks = [ks1, ks2, ks0, ks1, ks2, ks0]
    for r in range(5):
        for d in rot1 if r % 2 == 0 else rot2:
            x0 = (x0 + x1).astype(np.uint32)
            x1 = rotl(x1, d)
            x1 = (x1 ^ x0).astype(np.uint32)
        x0 = (x0 + ks[r]).astype(np.uint32)
        x1 = (x1 + ks[r + 1] + np.uint32(r + 1)).astype(np.uint32)
    return x0, x1


def _np_uniform_01(seed, n):
    key = (np.uint32((seed >> 32) & 0xFFFFFFFF), np.uint32(seed & 0xFFFFFFFF))
    i = np.arange(n, dtype=np.uint64)
    c1 = (i >> np.uint64(32)).astype(np.uint32)
    c2 = (i & np.uint64(0xFFFFFFFF)).astype(np.uint32)
    o0, o1 = _np_threefry2x32(key, c1, c2)
    bits = o0 ^ o1
    f = (((bits >> np.uint32(9)) | np.uint32(0x3F800000)).view(np.float32)
         - np.float32(1.0))
    return np.maximum(np.float32(0.0), f)


def _cand_indices() -> np.ndarray:
    perm = _np_uniform_01(42, _NUM_CLASSES)
    order = np.lexsort((np.arange(_NUM_CLASSES), -perm.astype(np.float64)))
    return order[:_NUM_SAMPLE].astype(np.int32)


_CAND = _cand_indices()

_CLASS_ID_TMPL = np.full((_TOTAL,), -1, np.int32)
_CLASS_ID_TMPL[:_NUM_SAMPLE] = _CAND
_CLASS_ID_TMPL[_ZERO_COL] = 0

_GIDX_TMPL = np.zeros((_TOTAL,), np.int32)
_GIDX_TMPL[:_NUM_SAMPLE] = _CAND
_GIDX_TMPL[_ZERO_COL] = 0

_NW = 32
_BPW = _TOTAL // _NW
_GCHUNK = 128
_NGC = _BPW // _GCHUNK
_PAIR_ROWS = _NUM_CLASSES // 2
_PAIR_W = 2 * _E


def _sc_gather_kernel(table_hbm, idx_hbm, out_hbm, idx_v, rows_v, sem):
    wid = lax.axis_index("s") * 2 + lax.axis_index("c")
    base = wid * _BPW
    for j in range(_NGC):
        pltpu.sync_copy(idx_hbm.at[pl.ds(base + j * _GCHUNK, _GCHUNK)],
                        idx_v.at[j])
    copies = []
    for j in range(_NGC):
        copies.append(
            pltpu.async_copy(
                table_hbm.at[idx_v.at[j]],
                rows_v.at[pl.ds(j * _GCHUNK, _GCHUNK)],
                sem,
            ))
    for c in copies:
        c.wait()
    pltpu.sync_copy(rows_v, out_hbm.at[pl.ds(base, _BPW)])


_TR_IN_W = 4096
_TR_TILES = _TR_IN_W // 128
_TR_OUT_R = _TR_TILES // 2 * 128
_PAIR_N = 500096
_TR_STEPS = (_NUM_CLASSES + _TR_IN_W - 1) // _TR_IN_W


def _tr_body(x_ref, out_ref):
    x = x_ref[...]
    nx = x / jnp.clip(jnp.sqrt(jnp.sum(x * x, axis=0, keepdims=True)),
                      1e-12, None)
    for t in range(_TR_TILES):
        blk = nx[:, t * 128:(t + 1) * 128].T
        r0 = (t // 2) * 128
        c0 = (t % 2) * _E
        out_ref[r0:r0 + 128, c0:c0 + _E] = blk


def _tc_transpose(wt):
    return pl.pallas_call(
        _tr_body,
        grid=(_TR_STEPS,),
        in_specs=[pl.BlockSpec((_E, _TR_IN_W), lambda k: (0, k))],
        out_specs=pl.BlockSpec((_TR_OUT_R, _PAIR_W), lambda k: (k, 0)),
        out_shape=jax.ShapeDtypeStruct((_PAIR_N, _PAIR_W), jnp.float32),
    )(wt)


def _sc_gather(table_pairs, pair_idx):
    mesh = plsc.VectorSubcoreMesh(core_axis_name="c", subcore_axis_name="s")
    kern = functools.partial(
        pl.kernel,
        mesh=mesh,
        out_type=jax.ShapeDtypeStruct((_TOTAL, _PAIR_W), jnp.float32),
        scratch_types=[
            pltpu.VMEM((_NGC, _GCHUNK), jnp.int32),
            pltpu.VMEM((_BPW, _PAIR_W), jnp.float32),
            pltpu.SemaphoreType.DMA,
        ],
    )(_sc_gather_kernel)
    return kern(table_pairs, pair_idx)


def _tc_body(emb_ref, w_ref, half_ref, cid_ref, labc_ref, labr_ref, out_ref,
             z_acc, marg, cnt):
    pid = pl.program_id(0)

    @pl.when(pid == 0)
    def _init():
        cnt[0, 0] = 0
        z_acc[...] = jnp.zeros_like(z_acc)

    labs_c = labc_ref[...]
    labs_r = labr_ref[...]
    cid = cid_ref[0]

    row_i = lax.broadcasted_iota(jnp.int32, (_BATCH, _CHUNK), 0)
    col_j = lax.broadcasted_iota(jnp.int32, (_BATCH, _CHUNK), 1)

    eq_ll = labs_c == labs_r
    dup_l = jnp.sum(jnp.where(eq_ll & (row_i < col_j), 1, 0),
                    axis=0, keepdims=True) > 0
    d = _BATCH - jnp.sum(dup_l.astype(jnp.int32))
    has0 = jnp.sum(jnp.where(labs_r == 0, 1, 0)) > 0
    include0 = jnp.logical_and(d < _BATCH, jnp.logical_not(has0))
    n_pos = d + include0.astype(jnp.int32)
    k_neg = _NUM_SAMPLE - n_pos

    gcol = pid * _CHUNK + lax.broadcasted_iota(jnp.int32, (1, _CHUNK), 1)
    region_c = gcol < _NUM_SAMPLE
    region_l = jnp.logical_and(gcol >= _LBL_OFF, gcol < _ZERO_COL)
    region_0 = gcol == _ZERO_COL

    eq = labs_c == cid
    in_p = jnp.sum(jnp.where(eq, 1, 0), axis=0, keepdims=True) > 0
    in_p = in_p | ((cid == 0) & include0)
    nonpos = (cid >= 0) & jnp.logical_not(in_p) & region_c
    npf = nonpos.astype(jnp.float32)
    tri = (row_i <= col_j).astype(jnp.float32)
    prefix_inc = lax.dot_general(npf, tri, (((1,), (0,)), ((), ())),
                                 preferred_element_type=jnp.float32)
    prefix_exc = prefix_inc - npf
    base = cnt[0, 0]
    sel = nonpos & ((base.astype(jnp.float32) + prefix_exc)
                    < k_neg.astype(jnp.float32))
    cnt[0, 0] = base + jnp.sum(npf).astype(jnp.int32)

    lbl_pos = gcol - _LBL_OFF
    dup_here = jnp.sum(jnp.where(eq & (row_i < lbl_pos), 1, 0),
                       axis=0, keepdims=True) > 0
    occ = jnp.logical_not(dup_here)

    colmask = sel | (region_l & occ) | (region_0 & (cid == 0) & include0)

    emb = emb_ref[...]
    en = jnp.sqrt(jnp.sum(emb * emb, axis=1, keepdims=True))
    nemb = emb / jnp.clip(en, 1e-12, None)
    wpair = w_ref[...]
    half = half_ref[...] > 0
    wn = jnp.where(half, wpair[:, _E:], wpair[:, :_E])

    logit = lax.dot_general(nemb, wn, (((1,), (1,)), ((), ())),
                            preferred_element_type=jnp.float32)
    logit = jnp.clip(logit, -1.0, 1.0)

    t = jnp.sum(nemb * wn, axis=1, keepdims=True)
    tcl = jnp.clip(jnp.clip(t, -1.0, 1.0), -1.0 + 1e-7, 1.0 - 1e-7)
    mrg = tcl * _COS_M - jnp.sqrt(1.0 - tcl * tcl) * _SIN_M

    @pl.when(pid == _LBL_OFF // _CHUNK)
    def _save_margin():
        marg[...] = mrg

    repl = eq & region_l & colmask
    ex = jnp.exp(_S * jnp.where(repl, mrg, logit))
    exm = jnp.where(colmask, ex, 0.0)
    z_acc[...] += jnp.sum(exm, axis=1, keepdims=True)

    @pl.when(pid == _NSTEPS - 1)
    def _finish():
        z = z_acc[...]
        m = marg[...]
        loss_vec = jnp.minimum(jnp.log(z) - _S * m, _NEG_LOG_CLIP)
        out_ref[...] = (jnp.sum(loss_vec) / float(_BATCH)).reshape(1, 1)


def _tc_compute(emb, rows, halfsel, class_id, labels):
    cid3 = class_id.reshape(_NSTEPS, 1, _CHUNK)
    half2 = halfsel.reshape(_TOTAL, 1)
    labs_c = labels.reshape(_BATCH, 1)
    labs_r = labels.reshape(1, _BATCH)
    return pl.pallas_call(
        _tc_body,
        grid=(_NSTEPS,),
        in_specs=[
            pl.BlockSpec((_BATCH, _E), lambda i: (0, 0)),
            pl.BlockSpec((_CHUNK, _PAIR_W), lambda i: (i, 0)),
            pl.BlockSpec((_CHUNK, 1), lambda i: (i, 0)),
            pl.BlockSpec((1, 1, _CHUNK), lambda i: (i, 0, 0)),
            pl.BlockSpec((_BATCH, 1), lambda i: (0, 0)),
            pl.BlockSpec((1, _BATCH), lambda i: (0, 0)),
        ],
        out_specs=pl.BlockSpec((1, 1), lambda i: (0, 0)),
        out_shape=jax.ShapeDtypeStruct((1, 1), jnp.float32),
        scratch_shapes=[
            pltpu.VMEM((_BATCH, 1), jnp.float32),
            pltpu.VMEM((_BATCH, 1), jnp.float32),
            pltpu.SMEM((1, 1), jnp.int32),
        ],
    )(emb, rows, half2, cid3, labs_c, labs_r)


def kernel(local_embeddings, local_labels, weight):
    labels = local_labels.astype(jnp.int32)
    class_id = jnp.asarray(_CLASS_ID_TMPL).at[_LBL_OFF:_ZERO_COL].set(labels)
    gidx = jnp.asarray(_GIDX_TMPL).at[_LBL_OFF:_ZERO_COL].set(labels)
    pair_idx = (gidx // 256) * 128 + gidx % 128
    halfsel = (gidx // 128) % 2
    table_pairs = _tc_transpose(weight.T)
    rows = _sc_gather(table_pairs, pair_idx)
    out = _tc_compute(local_embeddings, rows, halfsel, class_id, labels)
    return out[0, 0]


# speedup vs baseline: 2.2044x; 1.1788x over previous
import functools

import numpy as np
import jax
import jax.numpy as jnp
from jax import lax
from jax.experimental import pallas as pl
from jax.experimental.pallas import tpu as pltpu
from jax.experimental.pallas import tpu_sc as plsc

_NUM_CLASSES = 1000000
_E = 64
_NUM_SAMPLE = 10000
_BATCH = 1024
_S = 64.0
_COS_M = float(np.cos(0.5))
_SIN_M = float(np.sin(0.5))
_NEG_LOG_CLIP = float(-np.log(1e-30))

_N_CPAD = 10240
_LBL_OFF = 10240
_ZERO_COL = 11264
_TOTAL = 12288
_CHUNK = 1024
_NSTEPS = _TOTAL // _CHUNK


def _np_threefry2x32(keypair, x0, x1):
    rot1 = (13, 15, 26, 6)
    rot2 = (17, 29, 16, 24)
    ks0, ks1 = keypair
    ks2 = np.uint32(ks0 ^ ks1 ^ np.uint32(0x1BD11BDA))
    x0 = (x0 + ks0).astype(np.uint32)
    x1 = (x1 + ks1).astype(np.uint32)

    def rotl(v, d):
        return ((v << np.uint32(d)) | (v >> np.uint32(32 - d))).astype(np.uint32)

    ks = [ks1, ks2, ks0, ks1, ks2, ks0]
    for r in range(5):
        for d in rot1 if r % 2 == 0 else rot2:
            x0 = (x0 + x1).astype(np.uint32)
            x1 = rotl(x1, d)
            x1 = (x1 ^ x0).astype(np.uint32)
        x0 = (x0 + ks[r]).astype(np.uint32)
        x1 = (x1 + ks[r + 1] + np.uint32(r + 1)).astype(np.uint32)
    return x0, x1


def _np_uniform_01(seed, n):
    key = (np.uint32((seed >> 32) & 0xFFFFFFFF), np.uint32(seed & 0xFFFFFFFF))
    i = np.arange(n, dtype=np.uint64)
    c1 = (i >> np.uint64(32)).astype(np.uint32)
    c2 = (i & np.uint64(0xFFFFFFFF)).astype(np.uint32)
    o0, o1 = _np_threefry2x32(key, c1, c2)
    bits = o0 ^ o1
    f = (((bits >> np.uint32(9)) | np.uint32(0x3F800000)).view(np.float32)
         - np.float32(1.0))
    return np.maximum(np.float32(0.0), f)


def _cand_indices() -> np.ndarray:
    perm = _np_uniform_01(42, _NUM_CLASSES)
    order = np.lexsort((np.arange(_NUM_CLASSES), -perm.astype(np.float64)))
    return order[:_NUM_SAMPLE].astype(np.int32)


_CAND = _cand_indices()

_CLASS_ID_TMPL = np.full((_TOTAL,), -1, np.int32)
_CLASS_ID_TMPL[:_NUM_SAMPLE] = _CAND
_CLASS_ID_TMPL[_ZERO_COL] = 0

_GIDX_TMPL = np.zeros((_TOTAL,), np.int32)
_GIDX_TMPL[:_NUM_SAMPLE] = _CAND
_GIDX_TMPL[_ZERO_COL] = 0

_NW = 32
_BPW = _TOTAL // _NW
_GCHUNK = 128
_NGC = _BPW // _GCHUNK
_PAIR_ROWS = _NUM_CLASSES // 2
_PAIR_W = 2 * _E


def _sc_gather_kernel(table_hbm, idx_hbm, out_hbm, idx_v, rows_v, sem):
    wid = lax.axis_index("s") * 2 + lax.axis_index("c")
    base = wid * _BPW
    pltpu.sync_copy(idx_hbm.at[pl.ds(base, _BPW)], idx_v)
    copies = []
    for j in range(_NGC):
        copies.append(
            pltpu.async_copy(
                table_hbm.at[idx_v.at[pl.ds(j * _GCHUNK, _GCHUNK)]],
                rows_v.at[pl.ds(j * _GCHUNK, _GCHUNK)],
                sem,
            ))
    for c in copies:
        c.wait()
    pltpu.sync_copy(rows_v, out_hbm.at[pl.ds(base, _BPW)])


_TR_IN_W = 8192
_TR_TILES = _TR_IN_W // 128
_TR_OUT_R = _TR_TILES // 2 * 128
_PAIR_N = 500096
_TR_STEPS = (_NUM_CLASSES + _TR_IN_W - 1) // _TR_IN_W


def _tr_body(x_ref, out_ref):
    x = x_ref[...]
    nx = x / jnp.clip(jnp.sqrt(jnp.sum(x * x, axis=0, keepdims=True)),
                      1e-12, None)
    for t in range(_TR_TILES):
        blk = nx[:, t * 128:(t + 1) * 128].T
        r0 = (t // 2) * 128
        c0 = (t % 2) * _E
        out_ref[r0:r0 + 128, c0:c0 + _E] = blk


def _tc_transpose(wt):
    return pl.pallas_call(
        _tr_body,
        grid=(_TR_STEPS,),
        in_specs=[pl.BlockSpec((_E, _TR_IN_W), lambda k: (0, k))],
        out_specs=pl.BlockSpec((_TR_OUT_R, _PAIR_W), lambda k: (k, 0)),
        out_shape=jax.ShapeDtypeStruct((_PAIR_N, _PAIR_W), jnp.float32),
    )(wt)


def _sc_gather(table_pairs, pair_idx):
    mesh = plsc.VectorSubcoreMesh(core_axis_name="c", subcore_axis_name="s")
    kern = functools.partial(
        pl.kernel,
        mesh=mesh,
        out_type=jax.ShapeDtypeStruct((_TOTAL, _PAIR_W), jnp.float32),
        scratch_types=[
            pltpu.VMEM((_BPW,), jnp.int32),
            pltpu.VMEM((_BPW, _PAIR_W), jnp.float32),
            pltpu.SemaphoreType.DMA,
        ],
    )(_sc_gather_kernel)
    return kern(table_pairs, pair_idx)


def _tc_body(emb_ref, w_ref, half_ref, cid_ref, labc_ref, labr_ref, out_ref,
             z_acc, marg, cnt):
    pid = pl.program_id(0)

    @pl.when(pid == 0)
    def _init():
        cnt[0, 0] = 0
        z_acc[...] = jnp.zeros_like(z_acc)

    labs_c = labc_ref[...]
    labs_r = labr_ref[...]
    cid = cid_ref[0]

    row_i = lax.broadcasted_iota(jnp.int32, (_BATCH, _CHUNK), 0)
    col_j = lax.broadcasted_iota(jnp.int32, (_BATCH, _CHUNK), 1)

    eq_ll = labs_c == labs_r
    dup_l = jnp.sum(jnp.where(eq_ll & (row_i < col_j), 1, 0),
                    axis=0, keepdims=True) > 0
    d = _BATCH - jnp.sum(dup_l.astype(jnp.int32))
    has0 = jnp.sum(jnp.where(labs_r == 0, 1, 0)) > 0
    include0 = jnp.logical_and(d < _BATCH, jnp.logical_not(has0))
    n_pos = d + include0.astype(jnp.int32)
    k_neg = _NUM_SAMPLE - n_pos

    gcol = pid * _CHUNK + lax.broadcasted_iota(jnp.int32, (1, _CHUNK), 1)
    region_c = gcol < _NUM_SAMPLE
    region_l = jnp.logical_and(gcol >= _LBL_OFF, gcol < _ZERO_COL)
    region_0 = gcol == _ZERO_COL

    eq = labs_c == cid
    in_p = jnp.sum(jnp.where(eq, 1, 0), axis=0, keepdims=True) > 0
    in_p = in_p | ((cid == 0) & include0)
    nonpos = (cid >= 0) & jnp.logical_not(in_p) & region_c
    npf = nonpos.astype(jnp.float32)
    tri = (row_i <= col_j).astype(jnp.float32)
    prefix_inc = lax.dot_general(npf, tri, (((1,), (0,)), ((), ())),
                                 preferred_element_type=jnp.float32)
    prefix_exc = prefix_inc - npf
    base = cnt[0, 0]
    sel = nonpos & ((base.astype(jnp.float32) + prefix_exc)
                    < k_neg.astype(jnp.float32))
    cnt[0, 0] = base + jnp.sum(npf).astype(jnp.int32)

    lbl_pos = gcol - _LBL_OFF
    dup_here = jnp.sum(jnp.where(eq & (row_i < lbl_pos), 1, 0),
                       axis=0, keepdims=True) > 0
    occ = jnp.logical_not(dup_here)

    colmask = sel | (region_l & occ) | (region_0 & (cid == 0) & include0)

    emb = emb_ref[...]
    en = jnp.sqrt(jnp.sum(emb * emb, axis=1, keepdims=True))
    nemb = emb / jnp.clip(en, 1e-12, None)
    wpair = w_ref[...]
    half = half_ref[...] > 0
    wn = jnp.where(half, wpair[:, _E:], wpair[:, :_E])

    logit = lax.dot_general(nemb, wn, (((1,), (1,)), ((), ())),
                            preferred_element_type=jnp.float32)
    logit = jnp.clip(logit, -1.0, 1.0)

    t = jnp.sum(nemb * wn, axis=1, keepdims=True)
    tcl = jnp.clip(jnp.clip(t, -1.0, 1.0), -1.0 + 1e-7, 1.0 - 1e-7)
    mrg = tcl * _COS_M - jnp.sqrt(1.0 - tcl * tcl) * _SIN_M

    @pl.when(pid == _LBL_OFF // _CHUNK)
    def _save_margin():
        marg[...] = mrg

    repl = eq & region_l & colmask
    ex = jnp.exp(_S * jnp.where(repl, mrg, logit))
    exm = jnp.where(colmask, ex, 0.0)
    z_acc[...] += jnp.sum(exm, axis=1, keepdims=True)

    @pl.when(pid == _NSTEPS - 1)
    def _finish():
        z = z_acc[...]
        m = marg[...]
        loss_vec = jnp.minimum(jnp.log(z) - _S * m, _NEG_LOG_CLIP)
        out_ref[...] = (jnp.sum(loss_vec) / float(_BATCH)).reshape(1, 1)


def _tc_compute(emb, rows, halfsel, class_id, labels):
    cid3 = class_id.reshape(_NSTEPS, 1, _CHUNK)
    half2 = halfsel.reshape(_TOTAL, 1)
    labs_c = labels.reshape(_BATCH, 1)
    labs_r = labels.reshape(1, _BATCH)
    return pl.pallas_call(
        _tc_body,
        grid=(_NSTEPS,),
        in_specs=[
            pl.BlockSpec((_BATCH, _E), lambda i: (0, 0)),
            pl.BlockSpec((_CHUNK, _PAIR_W), lambda i: (i, 0)),
            pl.BlockSpec((_CHUNK, 1), lambda i: (i, 0)),
            pl.BlockSpec((1, 1, _CHUNK), lambda i: (i, 0, 0)),
            pl.BlockSpec((_BATCH, 1), lambda i: (0, 0)),
            pl.BlockSpec((1, _BATCH), lambda i: (0, 0)),
        ],
        out_specs=pl.BlockSpec((1, 1), lambda i: (0, 0)),
        out_shape=jax.ShapeDtypeStruct((1, 1), jnp.float32),
        scratch_shapes=[
            pltpu.VMEM((_BATCH, 1), jnp.float32),
            pltpu.VMEM((_BATCH, 1), jnp.float32),
            pltpu.SMEM((1, 1), jnp.int32),
        ],
    )(emb, rows, half2, cid3, labs_c, labs_r)


def kernel(local_embeddings, local_labels, weight):
    labels = local_labels.astype(jnp.int32)
    class_id = jnp.asarray(_CLASS_ID_TMPL).at[_LBL_OFF:_ZERO_COL].set(labels)
    gidx = jnp.asarray(_GIDX_TMPL).at[_LBL_OFF:_ZERO_COL].set(labels)
    pair_idx = (gidx // 256) * 128 + gidx % 128
    halfsel = (gidx // 128) % 2
    table_pairs = _tc_transpose(weight.T)
    rows = _sc_gather(table_pairs, pair_idx)
    out = _tc_compute(local_embeddings, rows, halfsel, class_id, labels)
    return out[0, 0]


# speedup vs baseline: 2.2444x; 1.0182x over previous
import functools

import numpy as np
import jax
import jax.numpy as jnp
from jax import lax
from jax.experimental import pallas as pl
from jax.experimental.pallas import tpu as pltpu
from jax.experimental.pallas import tpu_sc as plsc

_NUM_CLASSES = 1000000
_E = 64
_NUM_SAMPLE = 10000
_BATCH = 1024
_S = 64.0
_COS_M = float(np.cos(0.5))
_SIN_M = float(np.sin(0.5))
_NEG_LOG_CLIP = float(-np.log(1e-30))

_N_CPAD = 10240
_LBL_OFF = 10240
_ZERO_COL = 11264
_TOTAL = 12288
_CHUNK = 1024
_NSTEPS = _TOTAL // _CHUNK


def _np_threefry2x32(keypair, x0, x1):
    rot1 = (13, 15, 26, 6)
    rot2 = (17, 29, 16, 24)
    ks0, ks1 = keypair
    ks2 = np.uint32(ks0 ^ ks1 ^ np.uint32(0x1BD11BDA))
    x0 = (x0 + ks0).astype(np.uint32)
    x1 = (x1 + ks1).astype(np.uint32)

    def rotl(v, d):
        return ((v << np.uint32(d)) | (v >> np.uint32(32 - d))).astype(np.uint32)

    ks = [ks1, ks2, ks0, ks1, ks2, ks0]
    for r in range(5):
        for d in rot1 if r % 2 == 0 else rot2:
            x0 = (x0 + x1).astype(np.uint32)
            x1 = rotl(x1, d)
            x1 = (x1 ^ x0).astype(np.uint32)
        x0 = (x0 + ks[r]).astype(np.uint32)
        x1 = (x1 + ks[r + 1] + np.uint32(r + 1)).astype(np.uint32)
    return x0, x1


def _np_uniform_01(seed, n):
    key = (np.uint32((seed >> 32) & 0xFFFFFFFF), np.uint32(seed & 0xFFFFFFFF))
    i = np.arange(n, dtype=np.uint64)
    c1 = (i >> np.uint64(32)).astype(np.uint32)
    c2 = (i & np.uint64(0xFFFFFFFF)).astype(np.uint32)
    o0, o1 = _np_threefry2x32(key, c1, c2)
    bits = o0 ^ o1
    f = (((bits >> np.uint32(9)) | np.uint32(0x3F800000)).view(np.float32)
         - np.float32(1.0))
    return np.maximum(np.float32(0.0), f)


def _cand_indices() -> np.ndarray:
    perm = _np_uniform_01(42, _NUM_CLASSES)
    order = np.lexsort((np.arange(_NUM_CLASSES), -perm.astype(np.float64)))
    return order[:_NUM_SAMPLE].astype(np.int32)


_CAND = _cand_indices()

_CLASS_ID_TMPL = np.full((_TOTAL,), -1, np.int32)
_CLASS_ID_TMPL[:_NUM_SAMPLE] = _CAND
_CLASS_ID_TMPL[_ZERO_COL] = 0

_GIDX_TMPL = np.zeros((_TOTAL,), np.int32)
_GIDX_TMPL[:_NUM_SAMPLE] = _CAND
_GIDX_TMPL[_ZERO_COL] = 0

_NW = 32
_BPW = _TOTAL // _NW
_GCHUNK = 128
_NGC = _BPW // _GCHUNK
_PAIR_ROWS = _NUM_CLASSES // 2
_PAIR_W = 2 * _E


def _sc_gather_kernel(table_hbm, idx_hbm, out_hbm, idx_v, rows_v, sem, sem2):
    wid = lax.axis_index("s") * 2 + lax.axis_index("c")
    base = wid * _BPW
    pltpu.sync_copy(idx_hbm.at[pl.ds(base, _BPW)], idx_v)
    copies = []
    for j in range(_NGC):
        copies.append(
            pltpu.async_copy(
                table_hbm.at[idx_v.at[pl.ds(j * _GCHUNK, _GCHUNK)]],
                rows_v.at[pl.ds(j * _GCHUNK, _GCHUNK)],
                sem,
            ))
    outs = []
    for j in range(_NGC):
        copies[j].wait()
        outs.append(
            pltpu.async_copy(
                rows_v.at[pl.ds(j * _GCHUNK, _GCHUNK)],
                out_hbm.at[pl.ds(base + j * _GCHUNK, _GCHUNK)],
                sem2,
            ))
    for o in outs:
        o.wait()


_TR_IN_W = 8192
_TR_TILES = _TR_IN_W // 128
_TR_OUT_R = _TR_TILES // 2 * 128
_PAIR_N = 500096
_TR_STEPS = (_NUM_CLASSES + _TR_IN_W - 1) // _TR_IN_W


def _tr_body(x_ref, out_ref):
    x = x_ref[...]
    nx = x / jnp.clip(jnp.sqrt(jnp.sum(x * x, axis=0, keepdims=True)),
                      1e-12, None)
    for t in range(_TR_TILES):
        blk = nx[:, t * 128:(t + 1) * 128].T
        r0 = (t // 2) * 128
        c0 = (t % 2) * _E
        out_ref[r0:r0 + 128, c0:c0 + _E] = blk


def _tc_transpose(wt):
    return pl.pallas_call(
        _tr_body,
        grid=(_TR_STEPS,),
        in_specs=[pl.BlockSpec((_E, _TR_IN_W), lambda k: (0, k))],
        out_specs=pl.BlockSpec((_TR_OUT_R, _PAIR_W), lambda k: (k, 0)),
        out_shape=jax.ShapeDtypeStruct((_PAIR_N, _PAIR_W), jnp.float32),
    )(wt)


def _sc_gather(table_pairs, pair_idx):
    mesh = plsc.VectorSubcoreMesh(core_axis_name="c", subcore_axis_name="s")
    kern = functools.partial(
        pl.kernel,
        mesh=mesh,
        out_type=jax.ShapeDtypeStruct((_TOTAL, _PAIR_W), jnp.float32),
        scratch_types=[
            pltpu.VMEM((_BPW,), jnp.int32),
            pltpu.VMEM((_BPW, _PAIR_W), jnp.float32),
            pltpu.SemaphoreType.DMA,
            pltpu.SemaphoreType.DMA,
        ],
    )(_sc_gather_kernel)
    return kern(table_pairs, pair_idx)


def _tc_body(emb_ref, w_ref, half_ref, cid_ref, labc_ref, labr_ref, out_ref,
             z_acc, marg, sca, cm):
    pid = pl.program_id(0)

    labs_c = labc_ref[...]
    labs_r = labr_ref[...]
    cid = cid_ref[0]

    @pl.when(pid == 0)
    def _init():
        z_acc[...] = jnp.zeros_like(z_acc)
        row_i = lax.broadcasted_iota(jnp.int32, (_BATCH, _BATCH), 0)
        col_j = lax.broadcasted_iota(jnp.int32, (_BATCH, _BATCH), 1)
        eq_ll = labs_c == labs_r
        dup_l = jnp.sum(jnp.where(eq_ll & (row_i < col_j), 1, 0),
                        axis=0, keepdims=True) > 0
        d = _BATCH - jnp.sum(dup_l.astype(jnp.int32))
        has0 = jnp.sum(jnp.where(labs_r == 0, 1, 0)) > 0
        include0 = jnp.logical_and(d < _BATCH, jnp.logical_not(has0))
        sca[0, 0] = 0
        sca[0, 1] = _NUM_SAMPLE - d - include0.astype(jnp.int32)
        sca[0, 2] = include0.astype(jnp.int32)

    k_neg = sca[0, 1]
    include0 = sca[0, 2] > 0
    eq = labs_c == cid

    @pl.when(pid < _LBL_OFF // _CHUNK)
    def _sel_candidates():
        row_i = lax.broadcasted_iota(jnp.int32, (_CHUNK, _CHUNK), 0)
        col_j = lax.broadcasted_iota(jnp.int32, (_CHUNK, _CHUNK), 1)
        in_p = jnp.sum(jnp.where(eq, 1, 0), axis=0, keepdims=True) > 0
        in_p = in_p | ((cid == 0) & include0)
        nonpos = (cid >= 0) & jnp.logical_not(in_p)
        npf = nonpos.astype(jnp.float32)
        tri = (row_i <= col_j).astype(jnp.float32)
        prefix_inc = lax.dot_general(npf, tri, (((1,), (0,)), ((), ())),
                                     preferred_element_type=jnp.float32)
        base = sca[0, 0]
        sel = nonpos & ((base.astype(jnp.float32) + prefix_inc - npf)
                        < k_neg.astype(jnp.float32))
        sca[0, 0] = base + jnp.sum(npf).astype(jnp.int32)
        cm[...] = sel.astype(jnp.float32)

    @pl.when(pid == _LBL_OFF // _CHUNK)
    def _sel_labels():
        row_i = lax.broadcasted_iota(jnp.int32, (_BATCH, _CHUNK), 0)
        col_j = lax.broadcasted_iota(jnp.int32, (1, _CHUNK), 1)
        dup_here = jnp.sum(jnp.where(eq & (row_i < col_j), 1, 0),
                           axis=0, keepdims=True) > 0
        cm[...] = jnp.logical_not(dup_here).astype(jnp.float32)

    @pl.when(pid == _NSTEPS - 1)
    def _sel_zero():
        cm[...] = ((cid == 0) & include0).astype(jnp.float32)

    colmask = cm[...] > 0.0

    emb = emb_ref[...]
    en = jnp.sqrt(jnp.sum(emb * emb, axis=1, keepdims=True))
    nemb = emb / jnp.clip(en, 1e-12, None)
    wpair = w_ref[...]
    half = half_ref[...] > 0
    wn = jnp.where(half, wpair[:, _E:], wpair[:, :_E])

    logit = lax.dot_general(nemb, wn, (((1,), (1,)), ((), ())),
                            preferred_element_type=jnp.float32)
    logit = jnp.clip(logit, -1.0, 1.0)

    @pl.when(pid == _LBL_OFF // _CHUNK)
    def _save_margin():
        t = jnp.sum(nemb * wn, axis=1, keepdims=True)
        tcl = jnp.clip(jnp.clip(t, -1.0, 1.0), -1.0 + 1e-7, 1.0 - 1e-7)
        marg[...] = tcl * _COS_M - jnp.sqrt(1.0 - tcl * tcl) * _SIN_M

    repl = eq & colmask
    ex = jnp.exp(_S * jnp.where(repl, marg[...], logit))
    exm = jnp.where(colmask, ex, 0.0)
    z_acc[...] += jnp.sum(exm, axis=1, keepdims=True)

    @pl.when(pid == _NSTEPS - 1)
    def _finish():
        z = z_acc[...]
        m = marg[...]
        loss_vec = jnp.minimum(jnp.log(z) - _S * m, _NEG_LOG_CLIP)
        out_ref[...] = (jnp.sum(loss_vec) / float(_BATCH)).reshape(1, 1)


def _tc_compute(emb, rows, halfsel, class_id, labels):
    cid3 = class_id.reshape(_NSTEPS, 1, _CHUNK)
    half2 = halfsel.reshape(_TOTAL, 1)
    labs_c = labels.reshape(_BATCH, 1)
    labs_r = labels.reshape(1, _BATCH)
    return pl.pallas_call(
        _tc_body,
        grid=(_NSTEPS,),
        in_specs=[
            pl.BlockSpec((_BATCH, _E), lambda i: (0, 0)),
            pl.BlockSpec((_CHUNK, _PAIR_W), lambda i: (i, 0)),
            pl.BlockSpec((_CHUNK, 1), lambda i: (i, 0)),
            pl.BlockSpec((1, 1, _CHUNK), lambda i: (i, 0, 0)),
            pl.BlockSpec((_BATCH, 1), lambda i: (0, 0)),
            pl.BlockSpec((1, _BATCH), lambda i: (0, 0)),
        ],
        out_specs=pl.BlockSpec((1, 1), lambda i: (0, 0)),
        out_shape=jax.ShapeDtypeStruct((1, 1), jnp.float32),
        scratch_shapes=[
            pltpu.VMEM((_BATCH, 1), jnp.float32),
            pltpu.VMEM((_BATCH, 1), jnp.float32),
            pltpu.SMEM((1, 4), jnp.int32),
            pltpu.VMEM((1, _CHUNK), jnp.float32),
        ],
    )(emb, rows, half2, cid3, labs_c, labs_r)


def kernel(local_embeddings, local_labels, weight):
    labels = local_labels.astype(jnp.int32)
    class_id = jnp.asarray(_CLASS_ID_TMPL).at[_LBL_OFF:_ZERO_COL].set(labels)
    gidx = jnp.asarray(_GIDX_TMPL).at[_LBL_OFF:_ZERO_COL].set(labels)
    pair_idx = (gidx // 256) * 128 + gidx % 128
    halfsel = (gidx // 128) % 2
    table_pairs = _tc_transpose(weight.T)
    rows = _sc_gather(table_pairs, pair_idx)
    out = _tc_compute(local_embeddings, rows, halfsel, class_id, labels)
    return out[0, 0]


# speedup vs baseline: 2.4618x; 1.0969x over previous
import functools

import numpy as np
import jax
import jax.numpy as jnp
from jax import lax
from jax.experimental import pallas as pl
from jax.experimental.pallas import tpu as pltpu
from jax.experimental.pallas import tpu_sc as plsc

_NUM_CLASSES = 1000000
_E = 64
_NUM_SAMPLE = 10000
_BATCH = 1024
_S = 64.0
_COS_M = float(np.cos(0.5))
_SIN_M = float(np.sin(0.5))
_NEG_LOG_CLIP = float(-np.log(1e-30))

_N_CPAD = 10240
_LBL_OFF = 10240
_ZERO_COL = 11264
_TOTAL = 12288
_CHUNK = 1024
_NSTEPS = _TOTAL // _CHUNK


def _np_threefry2x32(keypair, x0, x1):
    rot1 = (13, 15, 26, 6)
    rot2 = (17, 29, 16, 24)
    ks0, ks1 = keypair
    ks2 = np.uint32(ks0 ^ ks1 ^ np.uint32(0x1BD11BDA))
    x0 = (x0 + ks0).astype(np.uint32)
    x1 = (x1 + ks1).astype(np.uint32)

    def rotl(v, d):
        return ((v << np.uint32(d)) | (v >> np.uint32(32 - d))).astype(np.uint32)

    ks = [ks1, ks2, ks0, ks1, ks2, ks0]
    for r in range(5):
        for d in rot1 if r % 2 == 0 else rot2:
            x0 = (x0 + x1).astype(np.uint32)
            x1 = rotl(x1, d)
            x1 = (x1 ^ x0).astype(np.uint32)
        x0 = (x0 + ks[r]).astype(np.uint32)
        x1 = (x1 + ks[r + 1] + np.uint32(r + 1)).astype(np.uint32)
    return x0, x1


def _np_uniform_01(seed, n):
    key = (np.uint32((seed >> 32) & 0xFFFFFFFF), np.uint32(seed & 0xFFFFFFFF))
    i = np.arange(n, dtype=np.uint64)
    c1 = (i >> np.uint64(32)).astype(np.uint32)
    c2 = (i & np.uint64(0xFFFFFFFF)).astype(np.uint32)
    o0, o1 = _np_threefry2x32(key, c1, c2)
    bits = o0 ^ o1
    f = (((bits >> np.uint32(9)) | np.uint32(0x3F800000)).view(np.float32)
         - np.float32(1.0))
    return np.maximum(np.float32(0.0), f)


def _cand_indices() -> np.ndarray:
    perm = _np_uniform_01(42, _NUM_CLASSES)
    order = np.lexsort((np.arange(_NUM_CLASSES), -perm.astype(np.float64)))
    return order[:_NUM_SAMPLE].astype(np.int32)


_CAND = _cand_indices()

_CLASS_ID_TMPL = np.full((_TOTAL,), -1, np.int32)
_CLASS_ID_TMPL[:_NUM_SAMPLE] = _CAND
_CLASS_ID_TMPL[_ZERO_COL] = 0

_GIDX_TMPL = np.zeros((_TOTAL,), np.int32)
_GIDX_TMPL[:_NUM_SAMPLE] = _CAND
_GIDX_TMPL[_ZERO_COL] = 0

_NW = 32
_BPW = _TOTAL // _NW
_GCHUNK = 128
_NGC = _BPW // _GCHUNK
_PAIR_ROWS = _NUM_CLASSES // 2
_PAIR_W = 2 * _E


def _sc_gather_kernel(table_hbm, idx_hbm, out_hbm, idx_v, rows_v, sem, sem2):
    wid = lax.axis_index("s") * 2 + lax.axis_index("c")
    base = wid * _BPW
    pltpu.sync_copy(idx_hbm.at[pl.ds(base, _BPW)], idx_v)
    copies = []
    for j in range(_NGC):
        copies.append(
            pltpu.async_copy(
                table_hbm.at[idx_v.at[pl.ds(j * _GCHUNK, _GCHUNK)]],
                rows_v.at[pl.ds(j * _GCHUNK, _GCHUNK)],
                sem,
            ))
    outs = []
    for j in range(_NGC):
        copies[j].wait()
        outs.append(
            pltpu.async_copy(
                rows_v.at[pl.ds(j * _GCHUNK, _GCHUNK)],
                out_hbm.at[pl.ds(base + j * _GCHUNK, _GCHUNK)],
                sem2,
            ))
    for o in outs:
        o.wait()


_TR_IN_W = 16384
_TR_TILES = _TR_IN_W // 128
_TR_OUT_R = _TR_TILES // 2 * 128
_PAIR_N = 500096
_TR_STEPS = (_NUM_CLASSES + _TR_IN_W - 1) // _TR_IN_W


def _tr_body(x_ref, out_ref):
    x = x_ref[...]
    nx = x / jnp.clip(jnp.sqrt(jnp.sum(x * x, axis=0, keepdims=True)),
                      1e-12, None)
    for t in range(_TR_TILES):
        blk = nx[:, t * 128:(t + 1) * 128].T
        r0 = (t // 2) * 128
        c0 = (t % 2) * _E
        out_ref[r0:r0 + 128, c0:c0 + _E] = blk


def _tc_transpose(wt):
    return pl.pallas_call(
        _tr_body,
        grid=(_TR_STEPS,),
        in_specs=[pl.BlockSpec((_E, _TR_IN_W), lambda k: (0, k))],
        out_specs=pl.BlockSpec((_TR_OUT_R, _PAIR_W), lambda k: (k, 0)),
        out_shape=jax.ShapeDtypeStruct((_PAIR_N, _PAIR_W), jnp.float32),
    )(wt)


def _sc_gather(table_pairs, pair_idx):
    mesh = plsc.VectorSubcoreMesh(core_axis_name="c", subcore_axis_name="s")
    kern = functools.partial(
        pl.kernel,
        mesh=mesh,
        out_type=jax.ShapeDtypeStruct((_TOTAL, _PAIR_W), jnp.float32),
        scratch_types=[
            pltpu.VMEM((_BPW,), jnp.int32),
            pltpu.VMEM((_BPW, _PAIR_W), jnp.float32),
            pltpu.SemaphoreType.DMA,
            pltpu.SemaphoreType.DMA,
        ],
    )(_sc_gather_kernel)
    return kern(table_pairs, pair_idx)


def _tc_body(emb_ref, w_ref, half_ref, cid_ref, labc_ref, labr_ref, out_ref,
             z_acc, marg, sca, cm):
    pid = pl.program_id(0)

    labs_c = labc_ref[...]
    labs_r = labr_ref[...]
    cid = cid_ref[0]

    @pl.when(pid == 0)
    def _init():
        z_acc[...] = jnp.zeros_like(z_acc)
        row_i = lax.broadcasted_iota(jnp.int32, (_BATCH, _BATCH), 0)
        col_j = lax.broadcasted_iota(jnp.int32, (_BATCH, _BATCH), 1)
        eq_ll = labs_c == labs_r
        dup_l = jnp.sum(jnp.where(eq_ll & (row_i < col_j), 1, 0),
                        axis=0, keepdims=True) > 0
        d = _BATCH - jnp.sum(dup_l.astype(jnp.int32))
        has0 = jnp.sum(jnp.where(labs_r == 0, 1, 0)) > 0
        include0 = jnp.logical_and(d < _BATCH, jnp.logical_not(has0))
        sca[0, 0] = 0
        sca[0, 1] = _NUM_SAMPLE - d - include0.astype(jnp.int32)
        sca[0, 2] = include0.astype(jnp.int32)

    k_neg = sca[0, 1]
    include0 = sca[0, 2] > 0
    eq = labs_c == cid

    @pl.when(pid < _LBL_OFF // _CHUNK)
    def _sel_candidates():
        row_i = lax.broadcasted_iota(jnp.int32, (_CHUNK, _CHUNK), 0)
        col_j = lax.broadcasted_iota(jnp.int32, (_CHUNK, _CHUNK), 1)
        in_p = jnp.sum(jnp.where(eq, 1, 0), axis=0, keepdims=True) > 0
        in_p = in_p | ((cid == 0) & include0)
        nonpos = (cid >= 0) & jnp.logical_not(in_p)
        npf = nonpos.astype(jnp.float32)
        tri = (row_i <= col_j).astype(jnp.float32)
        prefix_inc = lax.dot_general(npf, tri, (((1,), (0,)), ((), ())),
                                     preferred_element_type=jnp.float32)
        base = sca[0, 0]
        sel = nonpos & ((base.astype(jnp.float32) + prefix_inc - npf)
                        < k_neg.astype(jnp.float32))
        sca[0, 0] = base + jnp.sum(npf).astype(jnp.int32)
        cm[...] = sel.astype(jnp.float32)

    @pl.when(pid == _LBL_OFF // _CHUNK)
    def _sel_labels():
        row_i = lax.broadcasted_iota(jnp.int32, (_BATCH, _CHUNK), 0)
        col_j = lax.broadcasted_iota(jnp.int32, (1, _CHUNK), 1)
        dup_here = jnp.sum(jnp.where(eq & (row_i < col_j), 1, 0),
                           axis=0, keepdims=True) > 0
        cm[...] = jnp.logical_not(dup_here).astype(jnp.float32)

    @pl.when(pid == _NSTEPS - 1)
    def _sel_zero():
        cm[...] = ((cid == 0) & include0).astype(jnp.float32)

    colmask = cm[...] > 0.0

    emb = emb_ref[...]
    en = jnp.sqrt(jnp.sum(emb * emb, axis=1, keepdims=True))
    nemb = emb / jnp.clip(en, 1e-12, None)
    wpair = w_ref[...]
    half = half_ref[...] > 0
    wn = jnp.where(half, wpair[:, _E:], wpair[:, :_E])

    logit = lax.dot_general(nemb, wn, (((1,), (1,)), ((), ())),
                            preferred_element_type=jnp.float32)
    logit = jnp.clip(logit, -1.0, 1.0)

    @pl.when(pid == _LBL_OFF // _CHUNK)
    def _save_margin():
        t = jnp.sum(nemb * wn, axis=1, keepdims=True)
        tcl = jnp.clip(jnp.clip(t, -1.0, 1.0), -1.0 + 1e-7, 1.0 - 1e-7)
        marg[...] = tcl * _COS_M - jnp.sqrt(1.0 - tcl * tcl) * _SIN_M

    repl = eq & colmask
    ex = jnp.exp(_S * jnp.where(repl, marg[...], logit))
    exm = jnp.where(colmask, ex, 0.0)
    z_acc[...] += jnp.sum(exm, axis=1, keepdims=True)

    @pl.when(pid == _NSTEPS - 1)
    def _finish():
        z = z_acc[...]
        m = marg[...]
        loss_vec = jnp.minimum(jnp.log(z) - _S * m, _NEG_LOG_CLIP)
        out_ref[...] = (jnp.sum(loss_vec) / float(_BATCH)).reshape(1, 1)


def _tc_compute(emb, rows, halfsel, class_id, labels):
    cid3 = class_id.reshape(_NSTEPS, 1, _CHUNK)
    half2 = halfsel.reshape(_TOTAL, 1)
    labs_c = labels.reshape(_BATCH, 1)
    labs_r = labels.reshape(1, _BATCH)
    return pl.pallas_call(
        _tc_body,
        grid=(_NSTEPS,),
        in_specs=[
            pl.BlockSpec((_BATCH, _E), lambda i: (0, 0)),
            pl.BlockSpec((_CHUNK, _PAIR_W), lambda i: (i, 0)),
            pl.BlockSpec((_CHUNK, 1), lambda i: (i, 0)),
            pl.BlockSpec((1, 1, _CHUNK), lambda i: (i, 0, 0)),
            pl.BlockSpec((_BATCH, 1), lambda i: (0, 0)),
            pl.BlockSpec((1, _BATCH), lambda i: (0, 0)),
        ],
        out_specs=pl.BlockSpec((1, 1), lambda i: (0, 0)),
        out_shape=jax.ShapeDtypeStruct((1, 1), jnp.float32),
        scratch_shapes=[
            pltpu.VMEM((_BATCH, 1), jnp.float32),
            pltpu.VMEM((_BATCH, 1), jnp.float32),
            pltpu.SMEM((1, 4), jnp.int32),
            pltpu.VMEM((1, _CHUNK), jnp.float32),
        ],
    )(emb, rows, half2, cid3, labs_c, labs_r)


def kernel(local_embeddings, local_labels, weight):
    labels = local_labels.astype(jnp.int32)
    class_id = jnp.asarray(_CLASS_ID_TMPL).at[_LBL_OFF:_ZERO_COL].set(labels)
    gidx = jnp.asarray(_GIDX_TMPL).at[_LBL_OFF:_ZERO_COL].set(labels)
    pair_idx = (gidx // 256) * 128 + gidx % 128
    halfsel = (gidx // 128) % 2
    table_pairs = _tc_transpose(weight.T)
    rows = _sc_gather(table_pairs, pair_idx)
    out = _tc_compute(local_embeddings, rows, halfsel, class_id, labels)
    return out[0, 0]


# speedup vs baseline: 2.5668x; 1.0426x over previous
import functools

import numpy as np
import jax
import jax.numpy as jnp
from jax import lax
from jax.experimental import pallas as pl
from jax.experimental.pallas import tpu as pltpu
from jax.experimental.pallas import tpu_sc as plsc

_NUM_CLASSES = 1000000
_E = 64
_NUM_SAMPLE = 10000
_BATCH = 1024
_S = 64.0
_COS_M = float(np.cos(0.5))
_SIN_M = float(np.sin(0.5))
_NEG_LOG_CLIP = float(-np.log(1e-30))

_N_CPAD = 10240
_LBL_OFF = 10240
_ZERO_COL = 11264
_TOTAL = 12288
_CHUNK = 1024
_NSTEPS = _TOTAL // _CHUNK


def _np_threefry2x32(keypair, x0, x1):
    rot1 = (13, 15, 26, 6)
    rot2 = (17, 29, 16, 24)
    ks0, ks1 = keypair
    ks2 = np.uint32(ks0 ^ ks1 ^ np.uint32(0x1BD11BDA))
    x0 = (x0 + ks0).astype(np.uint32)
    x1 = (x1 + ks1).astype(np.uint32)

    def rotl(v, d):
        return ((v << np.uint32(d)) | (v >> np.uint32(32 - d))).astype(np.uint32)

    ks = [ks1, ks2, ks0, ks1, ks2, ks0]
    for r in range(5):
        for d in rot1 if r % 2 == 0 else rot2:
            x0 = (x0 + x1).astype(np.uint32)
            x1 = rotl(x1, d)
            x1 = (x1 ^ x0).astype(np.uint32)
        x0 = (x0 + ks[r]).astype(np.uint32)
        x1 = (x1 + ks[r + 1] + np.uint32(r + 1)).astype(np.uint32)
    return x0, x1


def _np_uniform_01(seed, n):
    key = (np.uint32((seed >> 32) & 0xFFFFFFFF), np.uint32(seed & 0xFFFFFFFF))
    i = np.arange(n, dtype=np.uint64)
    c1 = (i >> np.uint64(32)).astype(np.uint32)
    c2 = (i & np.uint64(0xFFFFFFFF)).astype(np.uint32)
    o0, o1 = _np_threefry2x32(key, c1, c2)
    bits = o0 ^ o1
    f = (((bits >> np.uint32(9)) | np.uint32(0x3F800000)).view(np.float32)
         - np.float32(1.0))
    return np.maximum(np.float32(0.0), f)


def _cand_indices() -> np.ndarray:
    perm = _np_uniform_01(42, _NUM_CLASSES)
    order = np.lexsort((np.arange(_NUM_CLASSES), -perm.astype(np.float64)))
    return order[:_NUM_SAMPLE].astype(np.int32)


_CAND = _cand_indices()

_CLASS_ID_TMPL = np.full((_TOTAL,), -1, np.int32)
_CLASS_ID_TMPL[:_NUM_SAMPLE] = _CAND
_CLASS_ID_TMPL[_ZERO_COL] = 0

_GIDX_TMPL = np.zeros((_TOTAL,), np.int32)
_GIDX_TMPL[:_NUM_SAMPLE] = _CAND
_GIDX_TMPL[_ZERO_COL] = 0

_NW = 32
_BPW = _TOTAL // _NW
_GCHUNK = 128
_NGC = _BPW // _GCHUNK
_PAIR_ROWS = _NUM_CLASSES // 2
_PAIR_W = 2 * _E


def _sc_gather_kernel(table_hbm, idx_hbm, out_hbm, idx_v, rows_v, sem, sem2):
    wid = lax.axis_index("s") * 2 + lax.axis_index("c")
    base = wid * _BPW
    pltpu.sync_copy(idx_hbm.at[pl.ds(base, _BPW)], idx_v)
    copies = []
    for j in range(_NGC):
        copies.append(
            pltpu.async_copy(
                table_hbm.at[idx_v.at[pl.ds(j * _GCHUNK, _GCHUNK)]],
                rows_v.at[pl.ds(j * _GCHUNK, _GCHUNK)],
                sem,
            ))
    outs = []
    for j in range(_NGC):
        copies[j].wait()
        outs.append(
            pltpu.async_copy(
                rows_v.at[pl.ds(j * _GCHUNK, _GCHUNK)],
                out_hbm.at[pl.ds(base + j * _GCHUNK, _GCHUNK)],
                sem2,
            ))
    for o in outs:
        o.wait()


_TR_IN_W = 32768
_TR_TILES = _TR_IN_W // 128
_TR_OUT_R = _TR_TILES // 2 * 128
_PAIR_N = 500096
_TR_STEPS = (_NUM_CLASSES + _TR_IN_W - 1) // _TR_IN_W


def _tr_body(x_ref, out_ref):
    x = x_ref[...]
    nx = x / jnp.clip(jnp.sqrt(jnp.sum(x * x, axis=0, keepdims=True)),
                      1e-12, None)
    for t in range(_TR_TILES):
        blk = nx[:, t * 128:(t + 1) * 128].T
        r0 = (t // 2) * 128
        c0 = (t % 2) * _E
        out_ref[r0:r0 + 128, c0:c0 + _E] = blk


def _tc_transpose(wt):
    return pl.pallas_call(
        _tr_body,
        grid=(_TR_STEPS,),
        in_specs=[pl.BlockSpec((_E, _TR_IN_W), lambda k: (0, k))],
        out_specs=pl.BlockSpec((_TR_OUT_R, _PAIR_W), lambda k: (k, 0)),
        out_shape=jax.ShapeDtypeStruct((_PAIR_N, _PAIR_W), jnp.float32),
    )(wt)


def _sc_gather(table_pairs, pair_idx):
    mesh = plsc.VectorSubcoreMesh(core_axis_name="c", subcore_axis_name="s")
    kern = functools.partial(
        pl.kernel,
        mesh=mesh,
        out_type=jax.ShapeDtypeStruct((_TOTAL, _PAIR_W), jnp.float32),
        scratch_types=[
            pltpu.VMEM((_BPW,), jnp.int32),
            pltpu.VMEM((_BPW, _PAIR_W), jnp.float32),
            pltpu.SemaphoreType.DMA,
            pltpu.SemaphoreType.DMA,
        ],
    )(_sc_gather_kernel)
    return kern(table_pairs, pair_idx)


def _tc_body(emb_ref, w_ref, half_ref, cid_ref, labc_ref, labr_ref, out_ref,
             z_acc, marg, sca, cm):
    pid = pl.program_id(0)

    labs_c = labc_ref[...]
    labs_r = labr_ref[...]
    cid = cid_ref[0]

    @pl.when(pid == 0)
    def _init():
        z_acc[...] = jnp.zeros_like(z_acc)
        row_i = lax.broadcasted_iota(jnp.int32, (_BATCH, _BATCH), 0)
        col_j = lax.broadcasted_iota(jnp.int32, (_BATCH, _BATCH), 1)
        eq_ll = labs_c == labs_r
        dup_l = jnp.sum(jnp.where(eq_ll & (row_i < col_j), 1, 0),
                        axis=0, keepdims=True) > 0
        d = _BATCH - jnp.sum(dup_l.astype(jnp.int32))
        has0 = jnp.sum(jnp.where(labs_r == 0, 1, 0)) > 0
        include0 = jnp.logical_and(d < _BATCH, jnp.logical_not(has0))
        sca[0, 0] = 0
        sca[0, 1] = _NUM_SAMPLE - d - include0.astype(jnp.int32)
        sca[0, 2] = include0.astype(jnp.int32)

    k_neg = sca[0, 1]
    include0 = sca[0, 2] > 0
    eq = labs_c == cid

    @pl.when(pid < _LBL_OFF // _CHUNK)
    def _sel_candidates():
        row_i = lax.broadcasted_iota(jnp.int32, (_CHUNK, _CHUNK), 0)
        col_j = lax.broadcasted_iota(jnp.int32, (_CHUNK, _CHUNK), 1)
        in_p = jnp.sum(jnp.where(eq, 1, 0), axis=0, keepdims=True) > 0
        in_p = in_p | ((cid == 0) & include0)
        nonpos = (cid >= 0) & jnp.logical_not(in_p)
        npf = nonpos.astype(jnp.float32)
        tri = (row_i <= col_j).astype(jnp.float32)
        prefix_inc = lax.dot_general(npf, tri, (((1,), (0,)), ((), ())),
                                     preferred_element_type=jnp.float32)
        base = sca[0, 0]
        sel = nonpos & ((base.astype(jnp.float32) + prefix_inc - npf)
                        < k_neg.astype(jnp.float32))
        sca[0, 0] = base + jnp.sum(npf).astype(jnp.int32)
        cm[...] = sel.astype(jnp.float32)

    @pl.when(pid == _LBL_OFF // _CHUNK)
    def _sel_labels():
        row_i = lax.broadcasted_iota(jnp.int32, (_BATCH, _CHUNK), 0)
        col_j = lax.broadcasted_iota(jnp.int32, (1, _CHUNK), 1)
        dup_here = jnp.sum(jnp.where(eq & (row_i < col_j), 1, 0),
                           axis=0, keepdims=True) > 0
        cm[...] = jnp.logical_not(dup_here).astype(jnp.float32)

    @pl.when(pid == _NSTEPS - 1)
    def _sel_zero():
        cm[...] = ((cid == 0) & include0).astype(jnp.float32)

    colmask = cm[...] > 0.0

    emb = emb_ref[...]
    en = jnp.sqrt(jnp.sum(emb * emb, axis=1, keepdims=True))
    nemb = emb / jnp.clip(en, 1e-12, None)
    wpair = w_ref[...]
    half = half_ref[...] > 0
    wn = jnp.where(half, wpair[:, _E:], wpair[:, :_E])

    logit = lax.dot_general(nemb, wn, (((1,), (1,)), ((), ())),
                            preferred_element_type=jnp.float32)
    logit = jnp.clip(logit, -1.0, 1.0)

    @pl.when(pid == _LBL_OFF // _CHUNK)
    def _save_margin():
        t = jnp.sum(nemb * wn, axis=1, keepdims=True)
        tcl = jnp.clip(jnp.clip(t, -1.0, 1.0), -1.0 + 1e-7, 1.0 - 1e-7)
        marg[...] = tcl * _COS_M - jnp.sqrt(1.0 - tcl * tcl) * _SIN_M

    repl = eq & colmask
    ex = jnp.exp(_S * jnp.where(repl, marg[...], logit))
    exm = jnp.where(colmask, ex, 0.0)
    z_acc[...] += jnp.sum(exm, axis=1, keepdims=True)

    @pl.when(pid == _NSTEPS - 1)
    def _finish():
        z = z_acc[...]
        m = marg[...]
        loss_vec = jnp.minimum(jnp.log(z) - _S * m, _NEG_LOG_CLIP)
        out_ref[...] = (jnp.sum(loss_vec) / float(_BATCH)).reshape(1, 1)


def _tc_compute(emb, rows, halfsel, class_id, labels):
    cid3 = class_id.reshape(_NSTEPS, 1, _CHUNK)
    half2 = halfsel.reshape(_TOTAL, 1)
    labs_c = labels.reshape(_BATCH, 1)
    labs_r = labels.reshape(1, _BATCH)
    return pl.pallas_call(
        _tc_body,
        grid=(_NSTEPS,),
        in_specs=[
            pl.BlockSpec((_BATCH, _E), lambda i: (0, 0)),
            pl.BlockSpec((_CHUNK, _PAIR_W), lambda i: (i, 0)),
            pl.BlockSpec((_CHUNK, 1), lambda i: (i, 0)),
            pl.BlockSpec((1, 1, _CHUNK), lambda i: (i, 0, 0)),
            pl.BlockSpec((_BATCH, 1), lambda i: (0, 0)),
            pl.BlockSpec((1, _BATCH), lambda i: (0, 0)),
        ],
        out_specs=pl.BlockSpec((1, 1), lambda i: (0, 0)),
        out_shape=jax.ShapeDtypeStruct((1, 1), jnp.float32),
        scratch_shapes=[
            pltpu.VMEM((_BATCH, 1), jnp.float32),
            pltpu.VMEM((_BATCH, 1), jnp.float32),
            pltpu.SMEM((1, 4), jnp.int32),
            pltpu.VMEM((1, _CHUNK), jnp.float32),
        ],
    )(emb, rows, half2, cid3, labs_c, labs_r)


def kernel(local_embeddings, local_labels, weight):
    labels = local_labels.astype(jnp.int32)
    class_id = jnp.asarray(_CLASS_ID_TMPL).at[_LBL_OFF:_ZERO_COL].set(labels)
    gidx = jnp.asarray(_GIDX_TMPL).at[_LBL_OFF:_ZERO_COL].set(labels)
    pair_idx = (gidx // 256) * 128 + gidx % 128
    halfsel = (gidx // 128) % 2
    table_pairs = _tc_transpose(weight.T)
    rows = _sc_gather(table_pairs, pair_idx)
    out = _tc_compute(local_embeddings, rows, halfsel, class_id, labels)
    return out[0, 0]


# speedup vs baseline: 2.5669x; 1.0000x over previous
import functools

import numpy as np
import jax
import jax.numpy as jnp
from jax import lax
from jax.experimental import pallas as pl
from jax.experimental.pallas import tpu as pltpu
from jax.experimental.pallas import tpu_sc as plsc

_NUM_CLASSES = 1000000
_E = 64
_NUM_SAMPLE = 10000
_BATCH = 1024
_S = 64.0
_COS_M = float(np.cos(0.5))
_SIN_M = float(np.sin(0.5))
_NEG_LOG_CLIP = float(-np.log(1e-30))

_N_CPAD = 10240
_LBL_OFF = 10240
_ZERO_COL = 11264
_TOTAL = 12288
_CHUNK = 1024
_NSTEPS = _TOTAL // _CHUNK


def _np_threefry2x32(keypair, x0, x1):
    rot1 = (13, 15, 26, 6)
    rot2 = (17, 29, 16, 24)
    ks0, ks1 = keypair
    ks2 = np.uint32(ks0 ^ ks1 ^ np.uint32(0x1BD11BDA))
    x0 = (x0 + ks0).astype(np.uint32)
    x1 = (x1 + ks1).astype(np.uint32)

    def rotl(v, d):
        return ((v << np.uint32(d)) | (v >> np.uint32(32 - d))).astype(np.uint32)

    ks = [ks1, ks2, ks0, ks1, ks2, ks0]
    for r in range(5):
        for d in rot1 if r % 2 == 0 else rot2:
            x0 = (x0 + x1).astype(np.uint32)
            x1 = rotl(x1, d)
            x1 = (x1 ^ x0).astype(np.uint32)
        x0 = (x0 + ks[r]).astype(np.uint32)
        x1 = (x1 + ks[r + 1] + np.uint32(r + 1)).astype(np.uint32)
    return x0, x1


def _np_uniform_01(seed, n):
    key = (np.uint32((seed >> 32) & 0xFFFFFFFF), np.uint32(seed & 0xFFFFFFFF))
    i = np.arange(n, dtype=np.uint64)
    c1 = (i >> np.uint64(32)).astype(np.uint32)
    c2 = (i & np.uint64(0xFFFFFFFF)).astype(np.uint32)
    o0, o1 = _np_threefry2x32(key, c1, c2)
    bits = o0 ^ o1
    f = (((bits >> np.uint32(9)) | np.uint32(0x3F800000)).view(np.float32)
         - np.float32(1.0))
    return np.maximum(np.float32(0.0), f)


def _cand_indices() -> np.ndarray:
    perm = _np_uniform_01(42, _NUM_CLASSES)
    order = np.lexsort((np.arange(_NUM_CLASSES), -perm.astype(np.float64)))
    return order[:_NUM_SAMPLE].astype(np.int32)


_CAND = _cand_indices()

_CLASS_ID_TMPL = np.full((_TOTAL,), -1, np.int32)
_CLASS_ID_TMPL[:_NUM_SAMPLE] = _CAND
_CLASS_ID_TMPL[_ZERO_COL] = 0

_GIDX_TMPL = np.zeros((_TOTAL,), np.int32)
_GIDX_TMPL[:_NUM_SAMPLE] = _CAND
_GIDX_TMPL[_ZERO_COL] = 0

_NW = 32
_BPW = _TOTAL // _NW
_GCHUNK = 128
_NGC = _BPW // _GCHUNK
_PAIR_W = 2 * _E


def _sc_gather_kernel(table_hbm, idx_hbm, out_hbm, idx_v, rows_v, sem, sem2):
    wid = lax.axis_index("s") * 2 + lax.axis_index("c")
    base = wid * _BPW
    pltpu.sync_copy(idx_hbm.at[pl.ds(base, _BPW)], idx_v)
    copies = []
    for j in range(_NGC):
        copies.append(
            pltpu.async_copy(
                table_hbm.at[idx_v.at[pl.ds(j * _GCHUNK, _GCHUNK)]],
                rows_v.at[pl.ds(j * _GCHUNK, _GCHUNK)],
                sem,
            ))
    outs = []
    for j in range(_NGC):
        copies[j].wait()
        outs.append(
            pltpu.async_copy(
                rows_v.at[pl.ds(j * _GCHUNK, _GCHUNK)],
                out_hbm.at[pl.ds(base + j * _GCHUNK, _GCHUNK)],
                sem2,
            ))
    for o in outs:
        o.wait()


_TR_IN_W = 32768
_TR_TILES = _TR_IN_W // 128
_TR_OUT_R = _TR_TILES // 2 * 128
_PAIR_N = 500096
_TR_STEPS = (_NUM_CLASSES + _TR_IN_W - 1) // _TR_IN_W


def _tr_body(x_ref, out_ref):
    x = x_ref[...]
    nx = x / jnp.clip(jnp.sqrt(jnp.sum(x * x, axis=0, keepdims=True)),
                      1e-12, None)
    for t in range(_TR_TILES):
        blk = nx[:, t * 128:(t + 1) * 128].T
        r0 = (t // 2) * 128
        c0 = (t % 2) * _E
        out_ref[r0:r0 + 128, c0:c0 + _E] = blk


def _tc_transpose(wt):
    return pl.pallas_call(
        _tr_body,
        grid=(_TR_STEPS,),
        in_specs=[pl.BlockSpec((_E, _TR_IN_W), lambda k: (0, k))],
        out_specs=pl.BlockSpec((_TR_OUT_R, _PAIR_W), lambda k: (k, 0)),
        out_shape=jax.ShapeDtypeStruct((_PAIR_N, _PAIR_W), jnp.float32),
    )(wt)


def _sc_gather(table_pairs, pair_idx):
    mesh = plsc.VectorSubcoreMesh(core_axis_name="c", subcore_axis_name="s")
    kern = functools.partial(
        pl.kernel,
        mesh=mesh,
        out_type=jax.ShapeDtypeStruct((_TOTAL, _PAIR_W), jnp.float32),
        scratch_types=[
            pltpu.VMEM((_BPW,), jnp.int32),
            pltpu.VMEM((_BPW, _PAIR_W), jnp.float32),
            pltpu.SemaphoreType.DMA,
            pltpu.SemaphoreType.DMA,
        ],
    )(_sc_gather_kernel)
    return kern(table_pairs, pair_idx)


def _tc_body(emb_ref, w_ref, half_ref, cid_ref, labc_ref, labr_ref, out_ref,
             z_acc, marg, sca, cm):
    pid = pl.program_id(0)

    labs_c = labc_ref[...]
    labs_r = labr_ref[...]
    cid = cid_ref[0]

    @pl.when(pid == 0)
    def _init():
        z_acc[...] = jnp.zeros_like(z_acc)
        row_i = lax.broadcasted_iota(jnp.int32, (_BATCH, _BATCH), 0)
        col_j = lax.broadcasted_iota(jnp.int32, (_BATCH, _BATCH), 1)
        eq_ll = labs_c == labs_r
        dup_l = jnp.sum(jnp.where(eq_ll & (row_i < col_j), 1, 0),
                        axis=0, keepdims=True) > 0
        d = _BATCH - jnp.sum(dup_l.astype(jnp.int32))
        has0 = jnp.sum(jnp.where(labs_r == 0, 1, 0)) > 0
        include0 = jnp.logical_and(d < _BATCH, jnp.logical_not(has0))
        sca[0, 0] = 0
        sca[0, 1] = _NUM_SAMPLE - d - include0.astype(jnp.int32)
        sca[0, 2] = include0.astype(jnp.int32)

    k_neg = sca[0, 1]
    include0 = sca[0, 2] > 0
    eq = labs_c == cid

    @pl.when(pid < _LBL_OFF // _CHUNK)
    def _sel_candidates():
        row_i = lax.broadcasted_iota(jnp.int32, (_CHUNK, _CHUNK), 0)
        col_j = lax.broadcasted_iota(jnp.int32, (_CHUNK, _CHUNK), 1)
        in_p = jnp.sum(jnp.where(eq, 1, 0), axis=0, keepdims=True) > 0
        in_p = in_p | ((cid == 0) & include0)
        nonpos = (cid >= 0) & jnp.logical_not(in_p)
        npf = nonpos.astype(jnp.float32)
        tri = (row_i <= col_j).astype(jnp.float32)
        prefix_inc = lax.dot_general(npf, tri, (((1,), (0,)), ((), ())),
                                     preferred_element_type=jnp.float32)
        base = sca[0, 0]
        sel = nonpos & ((base.astype(jnp.float32) + prefix_inc - npf)
                        < k_neg.astype(jnp.float32))
        sca[0, 0] = base + jnp.sum(npf).astype(jnp.int32)
        cm[...] = sel.astype(jnp.float32)

    @pl.when(pid == _LBL_OFF // _CHUNK)
    def _sel_labels():
        row_i = lax.broadcasted_iota(jnp.int32, (_BATCH, _CHUNK), 0)
        col_j = lax.broadcasted_iota(jnp.int32, (1, _CHUNK), 1)
        dup_here = jnp.sum(jnp.where(eq & (row_i < col_j), 1, 0),
                           axis=0, keepdims=True) > 0
        cm[...] = jnp.logical_not(dup_here).astype(jnp.float32)

    @pl.when(pid == _NSTEPS - 1)
    def _sel_zero():
        cm[...] = ((cid == 0) & include0).astype(jnp.float32)

    colmask = cm[...] > 0.0

    emb = emb_ref[...]
    en = jnp.sqrt(jnp.sum(emb * emb, axis=1, keepdims=True))
    nemb = emb / jnp.clip(en, 1e-12, None)
    wpair = w_ref[...]
    half = half_ref[...] > 0
    wn = jnp.where(half, wpair[:, _E:], wpair[:, :_E])

    logit = lax.dot_general(nemb, wn, (((1,), (1,)), ((), ())),
                            preferred_element_type=jnp.float32)
    logit = jnp.clip(logit, -1.0, 1.0)

    @pl.when(pid == _LBL_OFF // _CHUNK)
    def _save_margin():
        t = jnp.sum(nemb * wn, axis=1, keepdims=True)
        tcl = jnp.clip(jnp.clip(t, -1.0, 1.0), -1.0 + 1e-7, 1.0 - 1e-7)
        marg[...] = tcl * _COS_M - jnp.sqrt(1.0 - tcl * tcl) * _SIN_M

    repl = eq & colmask
    ex = jnp.exp(_S * jnp.where(repl, marg[...], logit))
    exm = jnp.where(colmask, ex, 0.0)
    z_acc[...] += jnp.sum(exm, axis=1, keepdims=True)

    @pl.when(pid == _NSTEPS - 1)
    def _finish():
        z = z_acc[...]
        m = marg[...]
        loss_vec = jnp.minimum(jnp.log(z) - _S * m, _NEG_LOG_CLIP)
        out_ref[...] = (jnp.sum(loss_vec) / float(_BATCH)).reshape(1, 1)


def _tc_compute(emb, rows, halfsel, class_id, labels):
    cid3 = class_id.reshape(_NSTEPS, 1, _CHUNK)
    half2 = halfsel.reshape(_TOTAL, 1)
    labs_c = labels.reshape(_BATCH, 1)
    labs_r = labels.reshape(1, _BATCH)
    return pl.pallas_call(
        _tc_body,
        grid=(_NSTEPS,),
        in_specs=[
            pl.BlockSpec((_BATCH, _E), lambda i: (0, 0)),
            pl.BlockSpec((_CHUNK, _PAIR_W), lambda i: (i, 0)),
            pl.BlockSpec((_CHUNK, 1), lambda i: (i, 0)),
            pl.BlockSpec((1, 1, _CHUNK), lambda i: (i, 0, 0)),
            pl.BlockSpec((_BATCH, 1), lambda i: (0, 0)),
            pl.BlockSpec((1, _BATCH), lambda i: (0, 0)),
        ],
        out_specs=pl.BlockSpec((1, 1), lambda i: (0, 0)),
        out_shape=jax.ShapeDtypeStruct((1, 1), jnp.float32),
        scratch_shapes=[
            pltpu.VMEM((_BATCH, 1), jnp.float32),
            pltpu.VMEM((_BATCH, 1), jnp.float32),
            pltpu.SMEM((1, 4), jnp.int32),
            pltpu.VMEM((1, _CHUNK), jnp.float32),
        ],
    )(emb, rows, half2, cid3, labs_c, labs_r)


def kernel(local_embeddings, local_labels, weight):
    labels = local_labels.astype(jnp.int32)
    class_id = jnp.asarray(_CLASS_ID_TMPL).at[_LBL_OFF:_ZERO_COL].set(labels)
    gidx = jnp.asarray(_GIDX_TMPL).at[_LBL_OFF:_ZERO_COL].set(labels)
    pair_idx = (gidx // 256) * 128 + gidx % 128
    halfsel = (gidx // 128) % 2
    table_pairs = _tc_transpose(weight.T)
    rows = _sc_gather(table_pairs, pair_idx)
    out = _tc_compute(local_embeddings, rows, halfsel, class_id, labels)
    return out[0, 0]
